# Initial kernel scaffold; baseline (speedup 1.0000x reference)
import jax
import jax.numpy as jnp
from jax.experimental import pallas as pl


def kernel(features, indices, W, gamma, beta):
    raise NotImplementedError("write your pallas kernel here")


# speedup vs baseline: 8.2671x; 8.2671x over previous
import functools

import jax
import jax.numpy as jnp
from jax import lax
from jax.experimental import pallas as pl
from jax.experimental.pallas import tpu as pltpu
from jax.experimental.pallas import tpu_sc as plsc

N = 150000
IN_CH = 16
OUT_CH = 16
BB = 2
DD = 41
HH = 400
WW = 352
NGRID = BB * DD * HH * WW
K27 = 27

NC = 2
NS = 16
NWK = NC * NS
NPAD = 150016
CHUNK = NPAD // NWK
NGROUPS = CHUNK // 16

_OFFSETS = [(dz, dy, dx) for dz in (-1, 0, 1) for dy in (-1, 0, 1) for dx in (-1, 0, 1)]
_CENTER = 13

_mesh = plsc.VectorSubcoreMesh(
    core_axis_name="c", subcore_axis_name="s", num_cores=NC, num_subcores=NS)


def _worker_base():
  wid = lax.axis_index("s") * NC + lax.axis_index("c")
  return wid * CHUNK


def _sc_scatter_body(bcol, dcol, hcol, wcol, grid, cb, cd, ch, cw, lin_v, val_v, sem):
  base = _worker_base()
  pltpu.sync_copy(bcol.at[pl.ds(base, CHUNK)], cb)
  pltpu.sync_copy(dcol.at[pl.ds(base, CHUNK)], cd)
  pltpu.sync_copy(hcol.at[pl.ds(base, CHUNK)], ch)
  pltpu.sync_copy(wcol.at[pl.ds(base, CHUNK)], cw)

  def body(g, carry):
    sl = pl.ds(g * 16, 16)
    lin = ((cb[sl] * DD + cd[sl]) * HH + ch[sl]) * WW + cw[sl]
    gi = base + g * 16 + lax.iota(jnp.int32, 16)
    lin_v[sl] = lin
    val_v[sl] = jnp.where(gi < N, gi, 0)
    return carry

  lax.fori_loop(0, NGROUPS, body, 0)
  pltpu.async_copy(val_v, grid.at[lin_v], sem).wait()


def _sc_gather_body(bcol, dcol, hcol, wcol, h27, grid, out,
                    cb, cd, ch, cw, lin_v, nlin_v, cand_v, hidx_v, acc_v, sem):
  base = _worker_base()
  pltpu.sync_copy(bcol.at[pl.ds(base, CHUNK)], cb)
  pltpu.sync_copy(dcol.at[pl.ds(base, CHUNK)], cd)
  pltpu.sync_copy(hcol.at[pl.ds(base, CHUNK)], ch)
  pltpu.sync_copy(wcol.at[pl.ds(base, CHUNK)], cw)

  def body0(g, carry):
    sl = pl.ds(g * 16, 16)
    lin_v[sl] = ((cb[sl] * DD + cd[sl]) * HH + ch[sl]) * WW + cw[sl]
    gi = base + g * 16 + lax.iota(jnp.int32, 16)
    hidx_v[sl] = gi * K27 + _CENTER
    return carry

  lax.fori_loop(0, NGROUPS, body0, 0)
  pltpu.async_copy(h27.at[hidx_v], acc_v, sem).wait()

  def mk_body1(dz, dy, dx, ck):
    def body1(g, carry):
      sl = pl.ds(g * 16, 16)
      nd = cd[sl] + dz
      nh = ch[sl] + dy
      nw = cw[sl] + dx
      ok = (nd >= 0) & (nd < DD) & (nh >= 0) & (nh < HH) & (nw >= 0) & (nw < WW)
      nlin_v[sl] = jnp.where(ok, lin_v[sl] + ck, NGRID)
      return carry
    return body1

  def mk_body2(k):
    def body2(g, carry):
      sl = pl.ds(g * 16, 16)
      c = cand_v[sl]
      hidx_v[sl] = jnp.where(c >= 0, c, N) * K27 + k
      return carry
    return body2

  for k, (dz, dy, dx) in enumerate(_OFFSETS):
    if k == _CENTER:
      continue
    ck = (dz * HH + dy) * WW + dx
    lax.fori_loop(0, NGROUPS, mk_body1(dz, dy, dx, ck), 0)
    pltpu.async_copy(grid.at[nlin_v], cand_v, sem).wait()
    lax.fori_loop(0, NGROUPS, mk_body2(k), 0)
    pltpu.async_copy(h27.at[hidx_v], acc_v, sem, add=True).wait()

  pltpu.sync_copy(acc_v, out.at[pl.ds(base, CHUNK)])


_sc_params = pltpu.CompilerParams(use_tc_tiling_on_sc=False)

_sc_scatter = functools.partial(
    pl.kernel,
    out_type=(),
    mesh=_mesh,
    compiler_params=_sc_params,
    scratch_types=[
        pltpu.VMEM((CHUNK,), jnp.int32),
        pltpu.VMEM((CHUNK,), jnp.int32),
        pltpu.VMEM((CHUNK,), jnp.int32),
        pltpu.VMEM((CHUNK,), jnp.int32),
        pltpu.VMEM((CHUNK,), jnp.int32),
        pltpu.VMEM((CHUNK,), jnp.int32),
        pltpu.SemaphoreType.DMA,
    ],
)(_sc_scatter_body)

_sc_gather = functools.partial(
    pl.kernel,
    out_type=jax.ShapeDtypeStruct((NPAD, OUT_CH), jnp.float32),
    mesh=_mesh,
    compiler_params=_sc_params,
    scratch_types=[
        pltpu.VMEM((CHUNK,), jnp.int32),
        pltpu.VMEM((CHUNK,), jnp.int32),
        pltpu.VMEM((CHUNK,), jnp.int32),
        pltpu.VMEM((CHUNK,), jnp.int32),
        pltpu.VMEM((CHUNK,), jnp.int32),
        pltpu.VMEM((CHUNK,), jnp.int32),
        pltpu.VMEM((CHUNK,), jnp.int32),
        pltpu.VMEM((CHUNK,), jnp.int32),
        pltpu.VMEM((CHUNK, OUT_CH), jnp.float32),
        pltpu.SemaphoreType.DMA,
    ],
)(_sc_gather_body)


_MM_BLK = 2344


def _mm_body(f_ref, w_ref, o_ref):
  o_ref[...] = jnp.dot(f_ref[...], w_ref[...], preferred_element_type=jnp.float32)


_mm = pl.pallas_call(
    _mm_body,
    grid=(NPAD // _MM_BLK,),
    in_specs=[
        pl.BlockSpec((_MM_BLK, IN_CH), lambda i: (i, 0)),
        pl.BlockSpec((IN_CH, K27 * OUT_CH), lambda i: (0, 0)),
    ],
    out_specs=pl.BlockSpec((_MM_BLK, K27 * OUT_CH), lambda i: (i, 0)),
    out_shape=jax.ShapeDtypeStruct((NPAD, K27 * OUT_CH), jnp.float32),
)


NROWS = NPAD * OUT_CH // 128
NROWS_VALID = N * OUT_CH // 128


def _bn_body(x_ref, g_ref, b_ref, y_ref):
  x = x_ref[...]
  rid = lax.broadcasted_iota(jnp.int32, (NROWS, 128), 0)
  m = (rid < NROWS_VALID).astype(jnp.float32)
  xm = x * m
  s = jnp.sum(xm, axis=0, keepdims=True)
  q = jnp.sum(xm * x, axis=0, keepdims=True)
  s_fold = s
  q_fold = q
  for j in range(1, 8):
    s_fold = s_fold + jnp.roll(s, 16 * j, axis=1)
    q_fold = q_fold + jnp.roll(q, 16 * j, axis=1)
  mean = s_fold * (1.0 / N)
  var = q_fold * (1.0 / N) - mean * mean
  inv = lax.rsqrt(var + 1e-5)
  scale = inv * g_ref[...]
  shift = b_ref[...] - mean * scale
  y_ref[...] = jnp.maximum(x * scale + shift, 0.0)


_bn = pl.pallas_call(
    _bn_body,
    out_shape=jax.ShapeDtypeStruct((NROWS, 128), jnp.float32),
)


def kernel(features, indices, W, gamma, beta):
  fpad = jnp.concatenate(
      [features, jnp.zeros((NPAD - N, IN_CH), jnp.float32)], axis=0)
  ipad = jnp.concatenate(
      [indices, jnp.broadcast_to(indices[0:1], (NPAD - N, 4))], axis=0)
  bcol = ipad[:, 0]
  dcol = ipad[:, 1]
  hcol = ipad[:, 2]
  wcol = ipad[:, 3]
  wcat = jnp.transpose(W, (1, 0, 2)).reshape(IN_CH, K27 * OUT_CH)

  h = _mm(fpad, wcat)
  h27 = h.reshape(NPAD * K27, OUT_CH)

  grid_ref = jax.new_ref(jnp.full((NGRID + 1,), -1, jnp.int32))
  _sc_scatter(bcol, dcol, hcol, wcol, grid_ref)
  conv = _sc_gather(bcol, dcol, hcol, wcol, h27, grid_ref)

  y = _bn(conv.reshape(NROWS, 128), jnp.tile(gamma, 8).reshape(1, 128),
          jnp.tile(beta, 8).reshape(1, 128))
  return y.reshape(NPAD, OUT_CH)[:N]
